# baseline (speedup 1.0000x reference)
import functools

import jax
import jax.numpy as jnp
from jax import lax
from jax.experimental import pallas as pl
from jax.experimental.pallas import tpu as pltpu
from jax.experimental.pallas import tpu_sc as plsc

VOCAB = 256
D = 64
B = 4096
N = 200

NUM_CORES = 2
NUM_SUBCORES = 16
NW = NUM_CORES * NUM_SUBCORES

TOTAL = B * N
PER_W = TOTAL // NW
CHUNK = 512
NCHUNK = PER_W // CHUNK


def _scale_body(t_ref, o_ref):
    o_ref[...] = t_ref[...] * (D ** 0.5)


def _scale_table(table):
    return pl.pallas_call(
        _scale_body,
        out_shape=jax.ShapeDtypeStruct((VOCAB, D), jnp.float32),
    )(table)


def _sc_body(table_hbm, idx_hbm, out_hbm, idx_v, rows_v, gsem, ssem):
    wid = lax.axis_index("s") * NUM_CORES + lax.axis_index("c")
    base = wid * PER_W
    pltpu.sync_copy(idx_hbm.at[wid], idx_v)

    def start_gather(j, p):
        pltpu.async_copy(table_hbm.at[idx_v.at[j]], rows_v.at[p], gsem.at[p])

    def wait_gather(j, p):
        pltpu.make_async_copy(
            table_hbm.at[idx_v.at[j]], rows_v.at[p], gsem.at[p]
        ).wait()

    def start_scatter(j, p):
        pltpu.async_copy(
            rows_v.at[p], out_hbm.at[pl.ds(base + j * CHUNK, CHUNK)], ssem.at[p]
        )

    def wait_scatter(p):
        pltpu.make_async_copy(
            rows_v.at[p], out_hbm.at[pl.ds(base, CHUNK)], ssem.at[p]
        ).wait()

    start_gather(0, 0)

    def step(j, carry):
        p = lax.rem(j, 2)
        q = 1 - p
        nj = j + 1

        @pl.when(nj < NCHUNK)
        def _():
            @pl.when(nj >= 2)
            def _():
                wait_scatter(q)

            start_gather(nj, q)

        wait_gather(j, p)
        start_scatter(j, p)
        return carry

    lax.fori_loop(0, NCHUNK, step, 0)
    wait_scatter(0)
    wait_scatter(1)


@jax.jit
def kernel(indices, embedding_matrix):
    table = _scale_table(embedding_matrix.astype(jnp.float32))
    idx = indices.astype(jnp.int32).reshape(NW, NCHUNK, CHUNK)

    mesh = plsc.VectorSubcoreMesh(core_axis_name="c", subcore_axis_name="s")
    out = pl.kernel(
        _sc_body,
        out_type=jax.ShapeDtypeStruct((TOTAL, D), jnp.float32),
        mesh=mesh,
        compiler_params=pltpu.CompilerParams(use_tc_tiling_on_sc=False),
        scratch_types=[
            pltpu.VMEM((NCHUNK, CHUNK), jnp.int32),
            pltpu.VMEM((2, CHUNK, D), jnp.float32),
            pltpu.SemaphoreType.DMA((2,)),
            pltpu.SemaphoreType.DMA((2,)),
        ],
    )(table, idx)
    return out.reshape(B, N, D)


# speedup vs baseline: 1.7163x; 1.7163x over previous
import functools

import jax
import jax.numpy as jnp
from jax import lax
from jax.experimental import pallas as pl
from jax.experimental.pallas import tpu as pltpu
from jax.experimental.pallas import tpu_sc as plsc

VOCAB = 256
D = 64
B = 4096
N = 200

NUM_CORES = 2
NUM_SUBCORES = 16
NW = NUM_CORES * NUM_SUBCORES

TOTAL = B * N
PER_W = TOTAL // NW
CHUNK = 512
NCHUNK = PER_W // CHUNK


def _scale_body(t_ref, o_ref):
    o_ref[...] = t_ref[...] * (D ** 0.5)


def _scale_table(table):
    return pl.pallas_call(
        _scale_body,
        out_shape=jax.ShapeDtypeStruct((VOCAB, D), jnp.float32),
    )(table)


def _sc_body(table_hbm, idx_hbm, out_hbm, table_v, idx_v, rows_v, gsem, ssem):
    wid = lax.axis_index("s") * NUM_CORES + lax.axis_index("c")
    base = wid * PER_W
    sid = lax.axis_index("s")

    @pl.when(sid == 0)
    def _():
        pltpu.sync_copy(table_hbm, table_v)

    plsc.subcore_barrier()
    pltpu.sync_copy(idx_hbm.at[wid], idx_v)

    def start_gather(j, p):
        pltpu.async_copy(table_v.at[idx_v.at[j]], rows_v.at[p], gsem.at[p])

    def wait_gather(j, p):
        pltpu.make_async_copy(
            table_v.at[idx_v.at[j]], rows_v.at[p], gsem.at[p]
        ).wait()

    def start_scatter(j, p):
        pltpu.async_copy(
            rows_v.at[p], out_hbm.at[pl.ds(base + j * CHUNK, CHUNK)], ssem.at[p]
        )

    def wait_scatter(p):
        pltpu.make_async_copy(
            rows_v.at[p], out_hbm.at[pl.ds(base, CHUNK)], ssem.at[p]
        ).wait()

    start_gather(0, 0)

    def step(j, carry):
        p = lax.rem(j, 2)
        q = 1 - p
        nj = j + 1

        @pl.when(nj < NCHUNK)
        def _():
            @pl.when(nj >= 2)
            def _():
                wait_scatter(q)

            start_gather(nj, q)

        wait_gather(j, p)
        start_scatter(j, p)
        return carry

    lax.fori_loop(0, NCHUNK, step, 0)
    wait_scatter(0)
    wait_scatter(1)


@jax.jit
def kernel(indices, embedding_matrix):
    table = _scale_table(embedding_matrix.astype(jnp.float32))
    idx = indices.astype(jnp.int32).reshape(NW, NCHUNK, CHUNK)

    mesh = plsc.VectorSubcoreMesh(core_axis_name="c", subcore_axis_name="s")
    out = pl.kernel(
        _sc_body,
        out_type=jax.ShapeDtypeStruct((TOTAL, D), jnp.float32),
        mesh=mesh,
        compiler_params=pltpu.CompilerParams(use_tc_tiling_on_sc=False),
        scratch_types=[
            pltpu.VMEM_SHARED((VOCAB, D), jnp.float32),
            pltpu.VMEM((NCHUNK, CHUNK), jnp.int32),
            pltpu.VMEM((2, CHUNK, D), jnp.float32),
            pltpu.SemaphoreType.DMA((2,)),
            pltpu.SemaphoreType.DMA((2,)),
        ],
    )(table, idx)
    return out.reshape(B, N, D)
